# baseline (device time: 108350 ns/iter reference)
import jax
import jax.numpy as jnp
from jax import lax
from jax.experimental import pallas as pl
from jax.experimental.pallas import tpu as pltpu

N_DEV = 16
H = N_DEV // 2
S = 8


def kernel(x):
    m, n = x.shape
    chunk = m // N_DEV
    sub = chunk // S

    def body(x_ref, out_ref, ccw_buf, cw_buf,
             ccw_ssem, ccw_rsem, cw_ssem, cw_rsem,
             agcw_ssem, agcw_rsem, agccw_ssem, agccw_rsem):
        d = lax.axis_index("i")
        left = lax.rem(d + N_DEV - 1, N_DEV)
        right = lax.rem(d + 1, N_DEV)

        def cidx(i):
            return lax.rem(i + 2 * N_DEV, N_DEV)

        def xsub(i, s):
            return x_ref.at[pl.ds(cidx(i) * chunk + s * sub, sub), :]

        def rs_ccw(k, s):
            src = xsub(d - H, s) if k == 0 else ccw_buf.at[k, pl.ds(s * sub, sub), :]
            return pltpu.make_async_remote_copy(
                src_ref=src,
                dst_ref=ccw_buf.at[k + 1, pl.ds(s * sub, sub), :],
                send_sem=ccw_ssem.at[k, s],
                recv_sem=ccw_rsem.at[k + 1, s],
                device_id=(left,),
                device_id_type=pl.DeviceIdType.MESH,
            )

        def rs_cw(k, s):
            src = (
                xsub(d + H - 1, s)
                if k == 0
                else cw_buf.at[k, pl.ds(s * sub, sub), :]
            )
            return pltpu.make_async_remote_copy(
                src_ref=src,
                dst_ref=cw_buf.at[k + 1, pl.ds(s * sub, sub), :],
                send_sem=cw_ssem.at[k, s],
                recv_sem=cw_rsem.at[k + 1, s],
                device_id=(right,),
                device_id_type=pl.DeviceIdType.MESH,
            )

        def ag_cw(t, s):
            c = cidx(d - t)
            return pltpu.make_async_remote_copy(
                src_ref=out_ref.at[pl.ds(c * chunk + s * sub, sub), :],
                dst_ref=out_ref.at[pl.ds(c * chunk + s * sub, sub), :],
                send_sem=agcw_ssem.at[t, s],
                recv_sem=agcw_rsem.at[t, s],
                device_id=(right,),
                device_id_type=pl.DeviceIdType.MESH,
            )

        def ag_ccw(t, s):
            c = cidx(d + t)
            return pltpu.make_async_remote_copy(
                src_ref=out_ref.at[pl.ds(c * chunk + s * sub, sub), :],
                dst_ref=out_ref.at[pl.ds(c * chunk + s * sub, sub), :],
                send_sem=agccw_ssem.at[t, s],
                recv_sem=agccw_rsem.at[t, s],
                device_id=(left,),
                device_id_type=pl.DeviceIdType.MESH,
            )

        barrier_sem = pltpu.get_barrier_semaphore()
        for nbr in (left, right):
            pl.semaphore_signal(
                barrier_sem, inc=1,
                device_id=(nbr,), device_id_type=pl.DeviceIdType.MESH,
            )
        pl.semaphore_wait(barrier_sem, 2)

        for s in range(S):
            rs_ccw(0, s).start()
            rs_cw(0, s).start()

        for k in range(H):
            for s in range(S):
                rs_ccw(k, s).wait_recv()
                if k + 1 < H:
                    ccw_buf[k + 1, pl.ds(s * sub, sub), :] = (
                        ccw_buf[k + 1, pl.ds(s * sub, sub), :]
                        + xsub(d - H + k + 1, s)[:, :]
                    )
                    rs_ccw(k + 1, s).start()
                else:
                    out_ref[pl.ds(d * chunk + s * sub, sub), :] = (
                        ccw_buf[H, pl.ds(s * sub, sub), :]
                        + cw_buf[H - 1, pl.ds(s * sub, sub), :]
                        + xsub(d, s)[:, :]
                    )
                    ag_cw(0, s).start()
                    ag_ccw(0, s).start()
                if k < H - 1:
                    rs_cw(k, s).wait_recv()
                    if k < H - 2:
                        cw_buf[k + 1, pl.ds(s * sub, sub), :] = (
                            cw_buf[k + 1, pl.ds(s * sub, sub), :]
                            + xsub(d + H - 2 - k, s)[:, :]
                        )
                    if k + 1 < H - 1:
                        rs_cw(k + 1, s).start()

        for t in range(H):
            for s in range(S):
                ag_cw(t, s).wait_recv()
                if t + 1 < H:
                    ag_cw(t + 1, s).start()
                if t < H - 1:
                    ag_ccw(t, s).wait_recv()
                    if t + 1 < H - 1:
                        ag_ccw(t + 1, s).start()

        for k in range(H):
            for s in range(S):
                rs_ccw(k, s).wait_send()
                if k < H - 1:
                    rs_cw(k, s).wait_send()
        for t in range(H):
            for s in range(S):
                ag_cw(t, s).wait_send()
                if t < H - 1:
                    ag_ccw(t, s).wait_send()

    return pl.pallas_call(
        body,
        out_shape=jax.ShapeDtypeStruct((m, n), x.dtype),
        in_specs=[pl.BlockSpec(memory_space=pltpu.VMEM)],
        out_specs=pl.BlockSpec(memory_space=pltpu.VMEM),
        scratch_shapes=[
            pltpu.VMEM((H + 1, chunk, n), x.dtype),
            pltpu.VMEM((H, chunk, n), x.dtype),
            pltpu.SemaphoreType.DMA((H, S)),
            pltpu.SemaphoreType.DMA((H + 1, S)),
            pltpu.SemaphoreType.DMA((H, S)),
            pltpu.SemaphoreType.DMA((H, S)),
            pltpu.SemaphoreType.DMA((H, S)),
            pltpu.SemaphoreType.DMA((H, S)),
            pltpu.SemaphoreType.DMA((H, S)),
            pltpu.SemaphoreType.DMA((H, S)),
        ],
        compiler_params=pltpu.CompilerParams(collective_id=0),
    )(x)
